# Initial kernel scaffold; baseline (speedup 1.0000x reference)
import jax
import jax.numpy as jnp
from jax.experimental import pallas as pl


def kernel(features, coords, num_frames, weight, bias):
    raise NotImplementedError("write your pallas kernel here")


# speedup vs baseline: 8.4774x; 8.4774x over previous
import functools

import jax
import jax.numpy as jnp
from jax import lax
from jax.experimental import pallas as pl
from jax.experimental.pallas import tpu as pltpu
from jax.experimental.pallas import tpu_sc as plsc

D = 128
D3 = D * D * D
KV = 27
CIN = 32
COUT = 32
NC = 2
NS = 16
NW = NC * NS
L = 16


def _mesh():
    return plsc.VectorSubcoreMesh(
        core_axis_name="c", subcore_axis_name="s", num_cores=NC, num_subcores=NS
    )


def _build_table(keys_pad, n_real):
    npad = keys_pad.shape[0]
    kchunk = 2048
    assert npad % kchunk == 0
    nchunks = npad // kchunk
    SL = D3 // NW

    @functools.partial(
        pl.kernel,
        out_type=jax.ShapeDtypeStruct((D3,), jnp.int32),
        mesh=_mesh(),
        compiler_params=pltpu.CompilerParams(use_tc_tiling_on_sc=False, needs_layout_passes=False),
        scratch_types=[
            pltpu.VMEM((SL + L,), jnp.int32),
            pltpu.VMEM((kchunk,), jnp.int32),
        ],
    )
    def k(keys_hbm, table_hbm, tbl_v, kbuf):
        wid = lax.axis_index("s") * NC + lax.axis_index("c")
        base = wid * SL
        nfill = jnp.full((L,), n_real, jnp.int32)

        @pl.loop(0, SL, step=L)
        def _(i):
            tbl_v[pl.ds(i, L)] = nfill

        iot = lax.iota(jnp.int32, L)

        @pl.loop(0, nchunks)
        def _(c):
            pltpu.sync_copy(keys_hbm.at[pl.ds(c * kchunk, kchunk)], kbuf)

            @pl.loop(0, kchunk, step=L)
            def _(g):
                k16 = kbuf[pl.ds(g, L)]
                loc = k16 - base
                m = (loc >= 0) & (loc < SL)
                loc = jnp.where(m, loc, SL)
                ids = c * kchunk + g + iot
                plsc.store_scatter(tbl_v, [loc], ids)

        pltpu.sync_copy(tbl_v.at[pl.ds(0, SL)], table_hbm.at[pl.ds(base, SL)])

    return k(keys_pad)


def _matmul(x, w):
    m = x.shape[0]
    bm = 512
    assert m % bm == 0

    def body(x_ref, w_ref, o_ref):
        o_ref[...] = jnp.dot(
            x_ref[...], w_ref[...], preferred_element_type=jnp.float32
        )

    return pl.pallas_call(
        body,
        grid=(m // bm,),
        in_specs=[
            pl.BlockSpec((bm, CIN), lambda i: (i, 0)),
            pl.BlockSpec((CIN, KV * COUT), lambda i: (0, 0)),
        ],
        out_specs=pl.BlockSpec((bm, KV * COUT), lambda i: (i, 0)),
        out_shape=jax.ShapeDtypeStruct((m, KV * COUT), jnp.float32),
    )(x, w)


def _gather_sum(keys_pad, table, y2, bias, n_real):
    npad = keys_pad.shape[0]
    PW = npad // NW
    C = 64
    assert PW % C == 0
    nch = PW // C
    S = KV * C
    SP = ((S + 127) // 128) * 128
    NJ = SP // 128
    GPK = C // L
    mrows = y2.shape[0]

    @functools.partial(
        pl.kernel,
        out_type=jax.ShapeDtypeStruct((npad, COUT), jnp.float32),
        mesh=_mesh(),
        compiler_params=pltpu.CompilerParams(use_tc_tiling_on_sc=False, needs_layout_passes=False),
        scratch_types=[
            pltpu.VMEM((PW,), jnp.int32),
            pltpu.VMEM((NJ, 128), jnp.int32),
            pltpu.VMEM((NJ, 128), jnp.int32),
            pltpu.VMEM((NJ, 128), jnp.int32),
            pltpu.VMEM((NJ, 128), jnp.int32),
            pltpu.VMEM((SP, COUT), jnp.float32),
            pltpu.VMEM((C, COUT), jnp.float32),
            pltpu.VMEM((COUT,), jnp.float32),
            pltpu.SemaphoreType.DMA,
        ],
    )
    def k(keys_hbm, table_hbm, y_hbm, bias_hbm, out_hbm,
          kbuf, nkb, mkb, tvb, rib, yrows, outb, biasv, sem):
        wid = lax.axis_index("s") * NC + lax.axis_index("c")
        wbase = wid * PW
        pltpu.sync_copy(keys_hbm.at[pl.ds(wbase, PW)], kbuf)
        pltpu.sync_copy(bias_hbm, biasv)

        zi = jnp.zeros((L,), jnp.int32)
        @pl.loop(S, SP, step=L)
        def _(s):
            nkb[s // 128, pl.ds(s % 128, L)] = zi
            mkb[s // 128, pl.ds(s % 128, L)] = zi

        @pl.loop(0, nch)
        def _(c):
            cbase = c * C

            @pl.loop(0, KV * GPK)
            def _(g):
                kk = g // GPK
                dz = kk // 9 - 1
                dy = (kk // 3) % 3 - 1
                dx = kk % 3 - 1
                delta = (dz * D + dy) * D + dx
                key16 = kbuf[pl.ds(cbase + (g % GPK) * L, L)]
                x = key16 & (D - 1)
                y = (key16 >> 7) & (D - 1)
                z = key16 >> 14
                nx = x + dx
                ny = y + dy
                nz = z + dz
                m = ((nx >= 0) & (nx < D) & (ny >= 0) & (ny < D)
                     & (nz >= 0) & (nz < D))
                nkey = jnp.where(m, key16 + delta, 0)
                s = g * L
                nkb[s // 128, pl.ds(s % 128, L)] = nkey
                mkb[s // 128, pl.ds(s % 128, L)] = jnp.where(m, 1, 0)

            descs = [
                pltpu.async_copy(table_hbm.at[nkb.at[j]], tvb.at[j], sem)
                for j in range(NJ)
            ]
            for d in descs:
                d.wait()

            @pl.loop(0, SP // L)
            def _(g):
                j = g // 8
                col = (g % 8) * L
                tv = tvb[j, pl.ds(col, L)]
                mm = mkb[j, pl.ds(col, L)]
                v = n_real + (tv - n_real) * mm
                kk = jnp.minimum(g // GPK, KV - 1)
                row = v * KV + kk
                row = jnp.clip(row, 0, mrows - 1)
                rib[j, pl.ds(col, L)] = row

            descs = [
                pltpu.async_copy(
                    y_hbm.at[rib.at[j]], yrows.at[pl.ds(j * 128, 128)], sem
                )
                for j in range(NJ)
            ]
            for d in descs:
                d.wait()

            b0 = biasv[pl.ds(0, L)]
            b1 = biasv[pl.ds(L, L)]
            for pg in range(C // L):
                init = []
                for _r in range(L):
                    init.append(b0)
                    init.append(b1)

                def body(kk, acc):
                    new = []
                    for r in range(L):
                        s = kk * C + pg * L + r
                        h0 = yrows[s, pl.ds(0, L)]
                        h1 = yrows[s, pl.ds(L, L)]
                        new.append(acc[2 * r] + h0)
                        new.append(acc[2 * r + 1] + h1)
                    return tuple(new)

                acc = lax.fori_loop(0, KV, body, tuple(init))
                for r in range(L):
                    outb[pg * L + r, pl.ds(0, L)] = acc[2 * r]
                    outb[pg * L + r, pl.ds(L, L)] = acc[2 * r + 1]

            pltpu.sync_copy(outb, out_hbm.at[pl.ds(wbase + cbase, C)])

    return k(keys_pad, table, y2, bias)


def kernel(features, coords, num_frames, weight, bias):
    n = features.shape[0]
    del num_frames
    npad = ((n + NW * 64) // (NW * 64)) * (NW * 64)
    keys = (coords[:, 1] * D + coords[:, 2]) * D + coords[:, 3]
    keys = keys.astype(jnp.int32)
    keys_tb = jnp.concatenate(
        [keys, jnp.full((npad - n,), -1, jnp.int32)])
    table = _build_table(keys_tb, n)

    feats_mm = jnp.concatenate(
        [features, jnp.zeros((npad - n, CIN), jnp.float32)])
    wflat = weight.transpose(1, 0, 2).reshape(CIN, KV * COUT)
    y = _matmul(feats_mm, wflat)
    y2 = y.reshape(npad * KV, COUT)

    out = _gather_sum(keys_tb, table, y2, bias, n)
    return out[:n]


# speedup vs baseline: 22.2989x; 2.6304x over previous
import functools

import jax
import jax.numpy as jnp
from jax import lax
from jax.experimental import pallas as pl
from jax.experimental.pallas import tpu as pltpu
from jax.experimental.pallas import tpu_sc as plsc

D = 128
D3 = D * D * D
KV = 27
CIN = 32
COUT = 32
NC = 2
NS = 16
NW = NC * NS
L = 16

_SC_PARAMS = dict(
    compiler_params=pltpu.CompilerParams(
        use_tc_tiling_on_sc=False, needs_layout_passes=False
    ),
)


def _mesh():
    return plsc.VectorSubcoreMesh(
        core_axis_name="c", subcore_axis_name="s", num_cores=NC, num_subcores=NS
    )


def _build_table(keys_pad, n_real):
    npad = keys_pad.shape[0]
    kchunk = 2048
    assert npad % kchunk == 0
    nchunks = npad // kchunk
    SL = D3 // NW
    SW = SL // 32

    @functools.partial(
        pl.kernel,
        out_type=[
            jax.ShapeDtypeStruct((D3,), jnp.int32),
            jax.ShapeDtypeStruct((D3 // 32,), jnp.int32),
        ],
        mesh=_mesh(),
        scratch_types=[
            pltpu.VMEM((SL + L,), jnp.int32),
            pltpu.VMEM((kchunk,), jnp.int32),
            pltpu.VMEM((SW,), jnp.int32),
        ],
        **_SC_PARAMS,
    )
    def k(keys_hbm, table_hbm, bmp_hbm, tbl_v, kbuf, bmp_v):
        wid = lax.axis_index("s") * NC + lax.axis_index("c")
        base = wid * SL
        nfill = jnp.full((L,), n_real, jnp.int32)

        @pl.loop(0, SL, step=L)
        def _(i):
            tbl_v[pl.ds(i, L)] = nfill

        iot = lax.iota(jnp.int32, L)

        @pl.loop(0, nchunks)
        def _(c):
            pltpu.sync_copy(keys_hbm.at[pl.ds(c * kchunk, kchunk)], kbuf)

            @pl.loop(0, kchunk, step=L)
            def _(g):
                k16 = kbuf[pl.ds(g, L)]
                loc = k16 - base
                m = (loc >= 0) & (loc < SL)
                loc = jnp.where(m, loc, SL)
                ids = c * kchunk + g + iot
                plsc.store_scatter(tbl_v, [loc], ids)

        iot32 = iot * 32

        @pl.loop(0, SW, step=L)
        def _(w0):
            acc = jnp.zeros((L,), jnp.int32)
            for b in range(32):
                occ = plsc.load_gather(tbl_v, [w0 * 32 + iot32 + b])
                bitc = (1 << b) if b < 31 else -(1 << 31)
                acc = acc | jnp.where(occ != n_real, bitc, 0)
            bmp_v[pl.ds(w0, L)] = acc

        pltpu.sync_copy(tbl_v.at[pl.ds(0, SL)], table_hbm.at[pl.ds(base, SL)])
        pltpu.sync_copy(bmp_v, bmp_hbm.at[pl.ds(wid * SW, SW)])

    return k(keys_pad)


def _matmul(x, wflat, w13, bias2d):
    m = x.shape[0]
    bm = 512
    assert m % bm == 0

    def body(x_ref, w_ref, w13_ref, b_ref, y_ref, z_ref):
        xv = x_ref[...]
        y_ref[...] = jnp.dot(xv, w_ref[...], preferred_element_type=jnp.float32)
        z_ref[...] = (
            jnp.dot(xv, w13_ref[...], preferred_element_type=jnp.float32)
            + b_ref[...]
        )

    return pl.pallas_call(
        body,
        grid=(m // bm,),
        in_specs=[
            pl.BlockSpec((bm, CIN), lambda i: (i, 0)),
            pl.BlockSpec((CIN, KV * COUT), lambda i: (0, 0)),
            pl.BlockSpec((CIN, COUT), lambda i: (0, 0)),
            pl.BlockSpec((1, COUT), lambda i: (0, 0)),
        ],
        out_specs=[
            pl.BlockSpec((bm, KV * COUT), lambda i: (i, 0)),
            pl.BlockSpec((bm, COUT), lambda i: (i, 0)),
        ],
        out_shape=[
            jax.ShapeDtypeStruct((m, KV * COUT), jnp.float32),
            jax.ShapeDtypeStruct((m, COUT), jnp.float32),
        ],
    )(x, wflat, w13, bias2d)


def _gather_sum(keys_pad, table, bmp, y2, z, n_real):
    npad = keys_pad.shape[0]
    PW = npad // NW
    C = 128
    assert PW % C == 0
    nch = PW // C
    CAPR = 26 * C
    DUMP = CAPR + 128
    CAP = DUMP + L
    NBMAX = (CAPR + 127) // 128
    BW = D3 // 32

    @functools.partial(
        pl.kernel,
        out_type=[
            jax.ShapeDtypeStruct((npad, COUT), jnp.float32),
            jax.ShapeDtypeStruct((NW, nch, CAP), jnp.int32),
            jax.ShapeDtypeStruct((NW, nch, CAP), jnp.int32),
        ],
        mesh=_mesh(),
        scratch_types=[
            pltpu.VMEM((PW,), jnp.int32),
            pltpu.VMEM((BW,), jnp.int32),
            pltpu.VMEM((CAP,), jnp.int32),
            pltpu.VMEM((CAP,), jnp.int32),
            pltpu.VMEM((128,), jnp.int32),
            pltpu.VMEM((128, COUT), jnp.float32),
            pltpu.VMEM((C + 1, COUT), jnp.float32),
            pltpu.SMEM((32,), jnp.int32),
            pltpu.SemaphoreType.DMA,
        ],
        **_SC_PARAMS,
    )
    def k(keys_hbm, table_hbm, bmp_hbm, y_hbm, z_hbm,
          out_hbm, ck_hbm, ca_hbm,
          kbuf, bmp_v, cand_k, cand_a, tvb, ybuf, acc, nbuf, sem):
        wid = lax.axis_index("s") * NC + lax.axis_index("c")
        wbase = wid * PW
        pltpu.sync_copy(keys_hbm.at[pl.ds(wbase, PW)], kbuf)
        pltpu.sync_copy(bmp_hbm, bmp_v)
        iot = lax.iota(jnp.int32, L)

        @pl.loop(0, nch)
        def _(c):
            cbase = c * C

            def grp(pg, off):
                key16 = kbuf[pl.ds(cbase + pg * L, L)]
                x = key16 & (D - 1)
                y = (key16 >> 7) & (D - 1)
                zz = key16 >> 14
                pab = (pg * L + iot) * 32
                mxm = x >= 1
                mxp = x <= D - 2
                mym = y >= 1
                myp = y <= D - 2
                mzm = zz >= 1
                mzp = zz <= D - 2
                for kk in range(KV):
                    if kk == KV // 2:
                        continue
                    dz = kk // 9 - 1
                    dy = (kk // 3) % 3 - 1
                    dx = kk % 3 - 1
                    delta = (dz * D + dy) * D + dx
                    m = None
                    for cond, neg, pos in (
                        (dx, mxm, mxp), (dy, mym, myp), (dz, mzm, mzp)
                    ):
                        if cond < 0:
                            m = neg if m is None else (m & neg)
                        elif cond > 0:
                            m = pos if m is None else (m & pos)
                    nkey = jnp.where(m, key16 + delta, 0)
                    wvec = plsc.load_gather(bmp_v, [nkey >> 5])
                    bit = (wvec >> (nkey & 31)) & 1
                    found = m & (bit != 0)
                    fi = jnp.where(found, 1, 0)
                    cum = plsc.cumsum(fi)
                    tot = plsc.all_reduce_population_count(found)
                    pos_ = jnp.where(found, off + cum - 1, DUMP)
                    plsc.store_scatter(cand_k, [pos_], nkey)
                    plsc.store_scatter(cand_a, [pos_], pab + kk)
                    off = off + tot
                return off

            off = lax.fori_loop(0, C // L, grp, jnp.zeros((L,), jnp.int32))
            r_cnt = jnp.max(off)

            zpad = jnp.zeros((L,), jnp.int32)
            apad = jnp.full((L,), C * 32, jnp.int32)
            for j in range(8):
                pidx = r_cnt + j * L + iot
                plsc.store_scatter(cand_k, [pidx], zpad)
                plsc.store_scatter(cand_a, [pidx], apad)

            nbuf[c] = (r_cnt + 127) // 128
            pltpu.sync_copy(cand_k, ck_hbm.at[wid, c])
            pltpu.sync_copy(cand_a, ca_hbm.at[wid, c])

        @pl.loop(0, nch)
        def _(c):
            cbase = c * C
            pltpu.sync_copy(
                z_hbm.at[pl.ds(wbase + cbase, C)], acc.at[pl.ds(0, C)]
            )
            pltpu.sync_copy(ck_hbm.at[wid, c], cand_k)
            pltpu.sync_copy(ca_hbm.at[wid, c], cand_a)
            nb = nbuf[c]

            @pl.loop(0, NBMAX)
            def _(b):
                @pl.when(b < nb)
                def _():
                    bb = b * 128
                    descs = []
                    for cg in range(8):
                        idxv = cand_k[pl.ds(bb + cg * L, L)]
                        descs.append(pltpu.async_copy(
                            table_hbm.at[plsc.Indices(idxv)],
                            tvb.at[pl.ds(cg * L, L)], sem))
                    for d in descs:
                        d.wait()
                    descs = []
                    for cg in range(8):
                        tv = tvb[pl.ds(cg * L, L)]
                        a16 = cand_a[pl.ds(bb + cg * L, L)]
                        rowv = tv * KV + (a16 & 31)
                        descs.append(pltpu.async_copy(
                            y_hbm.at[plsc.Indices(rowv)],
                            ybuf.at[pl.ds(cg * L, L)], sem))
                    for d in descs:
                        d.wait()
                    for cg in range(8):
                        a16 = cand_a[pl.ds(bb + cg * L, L)]
                        dstp = a16 >> 5
                        rows = cg * L + iot
                        for h in range(COUT):
                            hv = jnp.full((L,), h, jnp.int32)
                            vals = plsc.load_gather(ybuf, [rows, hv])
                            plsc.addupdate_scatter(acc, [dstp, hv], vals)

            pltpu.sync_copy(
                acc.at[pl.ds(0, C)], out_hbm.at[pl.ds(wbase + cbase, C)]
            )

    return k(keys_pad, table, bmp, y2, z)[0]


def kernel(features, coords, num_frames, weight, bias):
    n = features.shape[0]
    del num_frames
    blk = NW * 128
    npad = ((n + blk) // blk) * blk
    keys = (coords[:, 1] * D + coords[:, 2]) * D + coords[:, 3]
    keys = keys.astype(jnp.int32)
    keys_tb = jnp.concatenate(
        [keys, jnp.full((npad - n,), -1, jnp.int32)])
    table, bmp = _build_table(keys_tb, n)

    feats_mm = jnp.concatenate(
        [features, jnp.zeros((npad - n, CIN), jnp.float32)])
    wflat = weight.transpose(1, 0, 2).reshape(CIN, KV * COUT)
    w13 = weight[KV // 2]
    y, z = _matmul(feats_mm, wflat, w13, bias.reshape(1, COUT))
    y2 = y.reshape(npad * KV, COUT)

    out = _gather_sum(keys_tb, table, bmp, y2, z, n)
    return out[:n]
